# Initial kernel scaffold; baseline (speedup 1.0000x reference)
import jax
import jax.numpy as jnp
from jax.experimental import pallas as pl


def kernel(x, keys, W_models, b_models, W_van, b_van, W_tanh, b_tanh):
    raise NotImplementedError("write your pallas kernel here")


# speedup vs baseline: 1.2564x; 1.2564x over previous
import jax
import jax.numpy as jnp
from jax import lax
from jax.experimental import pallas as pl

B = 16384
D = 128
C = 64
O = 10
K = 8
TB = 256


def _fused_kernel(x_ref, keys_ref, wm_ref, bm_ref, wv_ref, bv_ref,
                  wt_ref, bt_ref,
                  ens_out, tanh_out, van_out, cosd_out, knn_out):
    x = x_ref[...]
    norm = jnp.sqrt(jnp.sum(x * x, axis=1, keepdims=True))
    xn = x / jnp.maximum(norm, 1e-12)

    dn = (((1,), (1,)), ((), ()))
    cos = lax.dot_general(xn, keys_ref[...], dn,
                          preferred_element_type=jnp.float32)
    cosd_out[...] = 1.0 - cos

    ci = cos[:, None, :]
    cj = cos[:, :, None]
    iota_p = lax.broadcasted_iota(jnp.int32, (1, C, C), 2)
    iota_j = lax.broadcasted_iota(jnp.int32, (1, C, C), 1)
    beats = (ci > cj) | ((ci == cj) & (iota_p < iota_j))
    rank = jnp.sum(beats.astype(jnp.float32), axis=2)
    maskf = (rank < float(K)).astype(jnp.float32)
    knn_out[...] = maskf

    w = cos * maskf
    denom = jnp.sum(w, axis=1, keepdims=True)

    z = lax.dot_general(x, wm_ref[...], dn,
                        preferred_element_type=jnp.float32) + bm_ref[...]
    ens = jnp.tanh(z)
    ens3 = ens.reshape(TB, O, C)
    num = jnp.sum(ens3 * w[:, None, :], axis=2)
    ens_out[...] = num / denom

    lv = lax.dot_general(x, wv_ref[...], dn,
                         preferred_element_type=jnp.float32) + bv_ref[...]
    m = jnp.max(lv, axis=1, keepdims=True)
    sh = lv - m
    van_out[...] = sh - jnp.log(jnp.sum(jnp.exp(sh), axis=1, keepdims=True))

    lt = lax.dot_general(x, wt_ref[...], dn,
                         preferred_element_type=jnp.float32) + bt_ref[...]
    tanh_out[...] = jnp.tanh(lt)


@jax.jit
def kernel(x, keys, W_models, b_models, W_van, b_van, W_tanh, b_tanh):
    wm_perm = W_models.transpose(1, 0, 2).reshape(O * C, D)
    bm_perm = b_models.T.reshape(1, O * C)
    bv = b_van.reshape(1, O)
    bt = b_tanh.reshape(1, O)

    grid = (B // TB,)
    f32 = jnp.float32
    outs = pl.pallas_call(
        _fused_kernel,
        grid=grid,
        in_specs=[
            pl.BlockSpec((TB, D), lambda i: (i, 0)),
            pl.BlockSpec((C, D), lambda i: (0, 0)),
            pl.BlockSpec((O * C, D), lambda i: (0, 0)),
            pl.BlockSpec((1, O * C), lambda i: (0, 0)),
            pl.BlockSpec((O, D), lambda i: (0, 0)),
            pl.BlockSpec((1, O), lambda i: (0, 0)),
            pl.BlockSpec((O, D), lambda i: (0, 0)),
            pl.BlockSpec((1, O), lambda i: (0, 0)),
        ],
        out_specs=[
            pl.BlockSpec((TB, O), lambda i: (i, 0)),
            pl.BlockSpec((TB, O), lambda i: (i, 0)),
            pl.BlockSpec((TB, O), lambda i: (i, 0)),
            pl.BlockSpec((TB, C), lambda i: (i, 0)),
            pl.BlockSpec((TB, C), lambda i: (i, 0)),
        ],
        out_shape=[
            jax.ShapeDtypeStruct((B, O), f32),
            jax.ShapeDtypeStruct((B, O), f32),
            jax.ShapeDtypeStruct((B, O), f32),
            jax.ShapeDtypeStruct((B, C), f32),
            jax.ShapeDtypeStruct((B, C), f32),
        ],
    )(x, keys, wm_perm, bm_perm, W_van, bv, W_tanh, bt)
    ens_o, tanh_o, van_o, cosd_o, knn_o = outs
    return (ens_o, tanh_o, van_o, cosd_o, knn_o)


# speedup vs baseline: 2.1973x; 1.7489x over previous
import jax
import jax.numpy as jnp
from jax import lax
from jax.experimental import pallas as pl

B = 16384
D = 128
C = 64
O = 10
K = 8
TB = 256


def _fused_kernel(x_ref, keys_ref, wm_ref, bm_ref, wv_ref, bv_ref,
                  wt_ref, bt_ref,
                  ens_out, tanh_out, van_out, cosd_out, knn_out):
    x = x_ref[...]
    norm = jnp.sqrt(jnp.sum(x * x, axis=1, keepdims=True))
    xn = x / jnp.maximum(norm, 1e-12)

    dn = (((1,), (1,)), ((), ()))
    cos = lax.dot_general(xn, keys_ref[...], dn,
                          preferred_element_type=jnp.float32)
    cosd_out[...] = 1.0 - cos

    iota = lax.broadcasted_iota(jnp.int32, (TB, C), 1).astype(jnp.float32)
    work = cos
    maskf = jnp.zeros((TB, C), jnp.float32)
    for _ in range(K):
        m = jnp.max(work, axis=1, keepdims=True)
        cand = work == m
        idx = jnp.min(jnp.where(cand, iota, float(C)), axis=1, keepdims=True)
        sel = iota == idx
        maskf = maskf + sel.astype(jnp.float32)
        work = jnp.where(sel, -jnp.inf, work)
    knn_out[...] = maskf

    w = cos * maskf
    denom = jnp.sum(w, axis=1, keepdims=True)

    z = lax.dot_general(x, wm_ref[...], dn,
                        preferred_element_type=jnp.float32) + bm_ref[...]
    ens = jnp.tanh(z)
    ens3 = ens.reshape(TB, O, C)
    num = jnp.sum(ens3 * w[:, None, :], axis=2)
    ens_out[...] = num / denom

    lv = lax.dot_general(x, wv_ref[...], dn,
                         preferred_element_type=jnp.float32) + bv_ref[...]
    m = jnp.max(lv, axis=1, keepdims=True)
    sh = lv - m
    van_out[...] = sh - jnp.log(jnp.sum(jnp.exp(sh), axis=1, keepdims=True))

    lt = lax.dot_general(x, wt_ref[...], dn,
                         preferred_element_type=jnp.float32) + bt_ref[...]
    tanh_out[...] = jnp.tanh(lt)


@jax.jit
def kernel(x, keys, W_models, b_models, W_van, b_van, W_tanh, b_tanh):
    wm_perm = W_models.transpose(1, 0, 2).reshape(O * C, D)
    bm_perm = b_models.T.reshape(1, O * C)
    bv = b_van.reshape(1, O)
    bt = b_tanh.reshape(1, O)

    grid = (B // TB,)
    f32 = jnp.float32
    outs = pl.pallas_call(
        _fused_kernel,
        grid=grid,
        in_specs=[
            pl.BlockSpec((TB, D), lambda i: (i, 0)),
            pl.BlockSpec((C, D), lambda i: (0, 0)),
            pl.BlockSpec((O * C, D), lambda i: (0, 0)),
            pl.BlockSpec((1, O * C), lambda i: (0, 0)),
            pl.BlockSpec((O, D), lambda i: (0, 0)),
            pl.BlockSpec((1, O), lambda i: (0, 0)),
            pl.BlockSpec((O, D), lambda i: (0, 0)),
            pl.BlockSpec((1, O), lambda i: (0, 0)),
        ],
        out_specs=[
            pl.BlockSpec((TB, O), lambda i: (i, 0)),
            pl.BlockSpec((TB, O), lambda i: (i, 0)),
            pl.BlockSpec((TB, O), lambda i: (i, 0)),
            pl.BlockSpec((TB, C), lambda i: (i, 0)),
            pl.BlockSpec((TB, C), lambda i: (i, 0)),
        ],
        out_shape=[
            jax.ShapeDtypeStruct((B, O), f32),
            jax.ShapeDtypeStruct((B, O), f32),
            jax.ShapeDtypeStruct((B, O), f32),
            jax.ShapeDtypeStruct((B, C), f32),
            jax.ShapeDtypeStruct((B, C), f32),
        ],
    )(x, keys, wm_perm, bm_perm, W_van, bv, W_tanh, bt)
    ens_o, tanh_o, van_o, cosd_o, knn_o = outs
    return (ens_o, tanh_o, van_o, cosd_o, knn_o)


# speedup vs baseline: 3.6025x; 1.6395x over previous
import jax
import jax.numpy as jnp
from jax import lax
from jax.experimental import pallas as pl

B = 16384
D = 128
C = 64
O = 10
K = 8
TB = 256


def _fused_kernel(x_ref, keys_ref, wm_ref, bm_ref, sel_ref, wv_ref, bv_ref,
                  wt_ref, bt_ref,
                  ens_out, tanh_out, van_out, cosd_out, knn_out):
    x = x_ref[...]
    norm = jnp.sqrt(jnp.sum(x * x, axis=1, keepdims=True))
    xn = x / jnp.maximum(norm, 1e-12)

    dn = (((1,), (1,)), ((), ()))
    cos = lax.dot_general(xn, keys_ref[...], dn,
                          preferred_element_type=jnp.float32)
    cosd_out[...] = 1.0 - cos

    iota = lax.broadcasted_iota(jnp.int32, (TB, C), 1).astype(jnp.float32)
    work = cos
    maskf = jnp.zeros((TB, C), jnp.float32)
    for _ in range(K):
        m = jnp.max(work, axis=1, keepdims=True)
        cand = work == m
        idx = jnp.min(jnp.where(cand, iota, float(C)), axis=1, keepdims=True)
        sel = iota == idx
        maskf = maskf + sel.astype(jnp.float32)
        work = jnp.where(sel, -jnp.inf, work)
    knn_out[...] = maskf

    w = cos * maskf
    denom = jnp.sum(w, axis=1, keepdims=True)
    w_scaled = w / denom

    z = lax.dot_general(x, wm_ref[...], dn,
                        preferred_element_type=jnp.float32) + bm_ref[...]
    ens = jnp.tanh(z)
    w_tiled = jnp.concatenate([w_scaled] * O, axis=1)
    p = ens * w_tiled
    ens_out[...] = lax.dot_general(
        p, sel_ref[...], (((1,), (0,)), ((), ())),
        preferred_element_type=jnp.float32)

    lv = lax.dot_general(x, wv_ref[...], dn,
                         preferred_element_type=jnp.float32) + bv_ref[...]
    m = jnp.max(lv, axis=1, keepdims=True)
    sh = lv - m
    van_out[...] = sh - jnp.log(jnp.sum(jnp.exp(sh), axis=1, keepdims=True))

    lt = lax.dot_general(x, wt_ref[...], dn,
                         preferred_element_type=jnp.float32) + bt_ref[...]
    tanh_out[...] = jnp.tanh(lt)


@jax.jit
def kernel(x, keys, W_models, b_models, W_van, b_van, W_tanh, b_tanh):
    wm_perm = W_models.transpose(1, 0, 2).reshape(O * C, D)
    bm_perm = b_models.T.reshape(1, O * C)
    sel = (jnp.arange(O * C)[:, None] // C ==
           jnp.arange(O)[None, :]).astype(jnp.float32)
    bv = b_van.reshape(1, O)
    bt = b_tanh.reshape(1, O)

    grid = (B // TB,)
    f32 = jnp.float32
    outs = pl.pallas_call(
        _fused_kernel,
        grid=grid,
        in_specs=[
            pl.BlockSpec((TB, D), lambda i: (i, 0)),
            pl.BlockSpec((C, D), lambda i: (0, 0)),
            pl.BlockSpec((O * C, D), lambda i: (0, 0)),
            pl.BlockSpec((1, O * C), lambda i: (0, 0)),
            pl.BlockSpec((O * C, O), lambda i: (0, 0)),
            pl.BlockSpec((O, D), lambda i: (0, 0)),
            pl.BlockSpec((1, O), lambda i: (0, 0)),
            pl.BlockSpec((O, D), lambda i: (0, 0)),
            pl.BlockSpec((1, O), lambda i: (0, 0)),
        ],
        out_specs=[
            pl.BlockSpec((TB, O), lambda i: (i, 0)),
            pl.BlockSpec((TB, O), lambda i: (i, 0)),
            pl.BlockSpec((TB, O), lambda i: (i, 0)),
            pl.BlockSpec((TB, C), lambda i: (i, 0)),
            pl.BlockSpec((TB, C), lambda i: (i, 0)),
        ],
        out_shape=[
            jax.ShapeDtypeStruct((B, O), f32),
            jax.ShapeDtypeStruct((B, O), f32),
            jax.ShapeDtypeStruct((B, O), f32),
            jax.ShapeDtypeStruct((B, C), f32),
            jax.ShapeDtypeStruct((B, C), f32),
        ],
    )(x, keys, wm_perm, bm_perm, sel, W_van, bv, W_tanh, bt)
    ens_o, tanh_o, van_o, cosd_o, knn_o = outs
    return (ens_o, tanh_o, van_o, cosd_o, knn_o)


# speedup vs baseline: 3.7269x; 1.0345x over previous
import jax
import jax.numpy as jnp
from jax import lax
from jax.experimental import pallas as pl

B = 16384
D = 128
C = 64
O = 10
K = 8
TB = 256


def _fused_kernel(x_ref, keys_ref, wm_ref, bm_ref, sel_ref, wv_ref, bv_ref,
                  wt_ref, bt_ref,
                  ens_out, tanh_out, van_out, cosd_out, knn_out):
    x = x_ref[...]
    norm = jnp.sqrt(jnp.sum(x * x, axis=1, keepdims=True))
    xn = x / jnp.maximum(norm, 1e-12)

    dn = (((1,), (1,)), ((), ()))
    cos = lax.dot_general(xn, keys_ref[...], dn,
                          preferred_element_type=jnp.float32)
    cosd_out[...] = 1.0 - cos

    iota = lax.broadcasted_iota(jnp.int32, (TB, C), 1).astype(jnp.float32)
    work = cos
    maskf = jnp.zeros((TB, C), jnp.float32)
    for _ in range(K):
        m = jnp.max(work, axis=1, keepdims=True)
        cand = work == m
        idx = jnp.min(jnp.where(cand, iota, float(C)), axis=1, keepdims=True)
        sel = iota == idx
        maskf = maskf + sel.astype(jnp.float32)
        work = jnp.where(sel, -jnp.inf, work)
    knn_out[...] = maskf

    w = cos * maskf
    denom = jnp.sum(w, axis=1, keepdims=True)

    z = lax.dot_general(x, wm_ref[...], dn,
                        preferred_element_type=jnp.float32) + bm_ref[...]
    ens = jnp.tanh(z)
    w_tiled = jnp.concatenate([w] * O, axis=1)
    p = ens * w_tiled
    num = lax.dot_general(
        p, sel_ref[...], (((1,), (0,)), ((), ())),
        preferred_element_type=jnp.float32)
    ens_out[...] = num / denom

    lv = lax.dot_general(x, wv_ref[...], dn,
                         preferred_element_type=jnp.float32) + bv_ref[...]
    m = jnp.max(lv, axis=1, keepdims=True)
    sh = lv - m
    van_out[...] = sh - jnp.log(jnp.sum(jnp.exp(sh), axis=1, keepdims=True))

    lt = lax.dot_general(x, wt_ref[...], dn,
                         preferred_element_type=jnp.float32) + bt_ref[...]
    tanh_out[...] = jnp.tanh(lt)


@jax.jit
def kernel(x, keys, W_models, b_models, W_van, b_van, W_tanh, b_tanh):
    wm_perm = W_models.transpose(1, 0, 2).reshape(O * C, D)
    bm_perm = b_models.T.reshape(1, O * C)
    sel = (jnp.arange(O * C)[:, None] // C ==
           jnp.arange(O)[None, :]).astype(jnp.float32)
    bv = b_van.reshape(1, O)
    bt = b_tanh.reshape(1, O)

    grid = (B // TB,)
    f32 = jnp.float32
    outs = pl.pallas_call(
        _fused_kernel,
        grid=grid,
        in_specs=[
            pl.BlockSpec((TB, D), lambda i: (i, 0)),
            pl.BlockSpec((C, D), lambda i: (0, 0)),
            pl.BlockSpec((O * C, D), lambda i: (0, 0)),
            pl.BlockSpec((1, O * C), lambda i: (0, 0)),
            pl.BlockSpec((O * C, O), lambda i: (0, 0)),
            pl.BlockSpec((O, D), lambda i: (0, 0)),
            pl.BlockSpec((1, O), lambda i: (0, 0)),
            pl.BlockSpec((O, D), lambda i: (0, 0)),
            pl.BlockSpec((1, O), lambda i: (0, 0)),
        ],
        out_specs=[
            pl.BlockSpec((TB, O), lambda i: (i, 0)),
            pl.BlockSpec((TB, O), lambda i: (i, 0)),
            pl.BlockSpec((TB, O), lambda i: (i, 0)),
            pl.BlockSpec((TB, C), lambda i: (i, 0)),
            pl.BlockSpec((TB, C), lambda i: (i, 0)),
        ],
        out_shape=[
            jax.ShapeDtypeStruct((B, O), f32),
            jax.ShapeDtypeStruct((B, O), f32),
            jax.ShapeDtypeStruct((B, O), f32),
            jax.ShapeDtypeStruct((B, C), f32),
            jax.ShapeDtypeStruct((B, C), f32),
        ],
    )(x, keys, wm_perm, bm_perm, sel, W_van, bv, W_tanh, bt)
    ens_o, tanh_o, van_o, cosd_o, knn_o = outs
    return (ens_o, tanh_o, van_o, cosd_o, knn_o)
